# baseline (device time: 38819 ns/iter reference)
import jax
import jax.numpy as jnp
from jax import lax
from jax.experimental import pallas as pl
from jax.experimental.pallas import tpu as pltpu

CHUNKS = (32, 32, 64, 64, 64, 64, 64, 64, 32, 32)
NC = len(CHUNKS)
OFFS = tuple(sum(CHUNKS[:k]) for k in range(NC))


def kernel(partial, gamma):
    _, m_tot, d = partial.shape
    m_half = m_tot // 2
    m_q = m_half // 2
    assert sum(CHUNKS) == m_q
    p2d = partial.reshape(m_tot, d)
    g2d = gamma.reshape(1, d)

    def body(p_ref, g_ref, o_ref, loc_buf, xrecv_buf, xsend_buf, ystage,
             loc_sem, stage_sems, xsend_sems, xrecv_sems, ysend_sems,
             yrecv_sems, ostore_sems):
        my_x = lax.axis_index("x")
        my_y = lax.axis_index("y")
        my_z = lax.axis_index("z")
        xp = (1 - my_x, my_y, my_z)
        yp = (my_x, 1 - my_y, my_z)

        row_mine = my_x * m_half + my_y * m_q
        row_send = (1 - my_x) * m_half + my_y * m_q
        out_off = my_y * m_q

        stage = []
        for k in range(NC):
            s = pltpu.make_async_copy(
                p_ref.at[pl.ds(row_send + OFFS[k], CHUNKS[k]), :],
                xsend_buf.at[pl.ds(OFFS[k], CHUNKS[k]), :],
                stage_sems.at[k],
            )
            s.start()
            stage.append(s)
        loc_dma = pltpu.make_async_copy(
            p_ref.at[pl.ds(row_mine, m_q), :], loc_buf, loc_sem
        )
        loc_dma.start()

        barrier = pltpu.get_barrier_semaphore()
        for nbr in (xp, yp):
            pl.semaphore_signal(
                barrier, inc=1, device_id=nbr,
                device_id_type=pl.DeviceIdType.MESH,
            )
        pl.semaphore_wait(barrier, 2)

        xrdma = []
        for k in range(NC):
            stage[k].wait()
            r = pltpu.make_async_remote_copy(
                src_ref=xsend_buf.at[pl.ds(OFFS[k], CHUNKS[k]), :],
                dst_ref=xrecv_buf.at[pl.ds(OFFS[k], CHUNKS[k]), :],
                send_sem=xsend_sems.at[k],
                recv_sem=xrecv_sems.at[k],
                device_id=xp,
                device_id_type=pl.DeviceIdType.MESH,
            )
            r.start()
            xrdma.append(r)

        loc_dma.wait()

        yrdma = []
        ostore = []
        for k in range(NC):
            sl = pl.ds(OFFS[k], CHUNKS[k])
            xrdma[k].wait_recv()
            y = loc_buf[sl, :] + xrecv_buf[sl, :]
            ms = jnp.mean(y * y, axis=-1, keepdims=True)
            ystage[sl, :] = y * lax.rsqrt(ms + 1e-6) * g_ref[0, :][None, :]
            osl = pl.ds(out_off + OFFS[k], CHUNKS[k])
            r = pltpu.make_async_remote_copy(
                src_ref=ystage.at[sl, :],
                dst_ref=o_ref.at[osl, :],
                send_sem=ysend_sems.at[k],
                recv_sem=yrecv_sems.at[k],
                device_id=yp,
                device_id_type=pl.DeviceIdType.MESH,
            )
            r.start()
            yrdma.append(r)
            s = pltpu.make_async_copy(
                ystage.at[sl, :], o_ref.at[osl, :], ostore_sems.at[k]
            )
            s.start()
            ostore.append(s)

        for k in range(NC):
            yrdma[k].wait_recv()
            xrdma[k].wait_send()
            yrdma[k].wait_send()
            ostore[k].wait()

    out = pl.pallas_call(
        body,
        out_shape=jax.ShapeDtypeStruct((m_half, d), jnp.float32),
        in_specs=[
            pl.BlockSpec(memory_space=pl.ANY),
            pl.BlockSpec(memory_space=pltpu.VMEM),
        ],
        out_specs=pl.BlockSpec(memory_space=pl.ANY),
        scratch_shapes=[
            pltpu.VMEM((m_q, d), jnp.float32),
            pltpu.VMEM((m_q, d), jnp.float32),
            pltpu.VMEM((m_q, d), jnp.float32),
            pltpu.VMEM((m_q, d), jnp.float32),
            pltpu.SemaphoreType.DMA,
            pltpu.SemaphoreType.DMA((NC,)),
            pltpu.SemaphoreType.DMA((NC,)),
            pltpu.SemaphoreType.DMA((NC,)),
            pltpu.SemaphoreType.DMA((NC,)),
            pltpu.SemaphoreType.DMA((NC,)),
            pltpu.SemaphoreType.DMA((NC,)),
        ],
        compiler_params=pltpu.CompilerParams(collective_id=0),
    )(p2d, g2d)
    return out


# device time: 38115 ns/iter; 1.0185x vs baseline; 1.0185x over previous
import jax
import jax.numpy as jnp
from jax import lax
from jax.experimental import pallas as pl
from jax.experimental.pallas import tpu as pltpu

CHUNKS = (64, 64, 64, 64, 64, 64, 64, 64)
NC = len(CHUNKS)
OFFS = tuple(sum(CHUNKS[:k]) for k in range(NC))


def kernel(partial, gamma):
    _, m_tot, d = partial.shape
    m_half = m_tot // 2
    m_q = m_half // 2
    assert sum(CHUNKS) == m_q
    p2d = partial.reshape(m_tot, d)
    g2d = gamma.reshape(1, d)

    def body(p_ref, g_ref, o_ref, loc_buf, xrecv_buf, xsend_buf, ystage,
             loc_sem, stage_sems, xsend_sems, xrecv_sems, ysend_sems,
             yrecv_sems, ostore_sems):
        my_x = lax.axis_index("x")
        my_y = lax.axis_index("y")
        my_z = lax.axis_index("z")
        xp = (1 - my_x, my_y, my_z)
        yp = (my_x, 1 - my_y, my_z)

        row_mine = my_x * m_half + my_y * m_q
        row_send = (1 - my_x) * m_half + my_y * m_q
        out_off = my_y * m_q

        stage = []
        for k in range(NC):
            s = pltpu.make_async_copy(
                p_ref.at[pl.ds(row_send + OFFS[k], CHUNKS[k]), :],
                xsend_buf.at[pl.ds(OFFS[k], CHUNKS[k]), :],
                stage_sems.at[k],
            )
            s.start()
            stage.append(s)
        loc_dma = pltpu.make_async_copy(
            p_ref.at[pl.ds(row_mine, m_q), :], loc_buf, loc_sem
        )
        loc_dma.start()

        barrier = pltpu.get_barrier_semaphore()
        for nbr in (xp, yp):
            pl.semaphore_signal(
                barrier, inc=1, device_id=nbr,
                device_id_type=pl.DeviceIdType.MESH,
            )
        pl.semaphore_wait(barrier, 2)

        xrdma = []
        for k in range(NC):
            stage[k].wait()
            r = pltpu.make_async_remote_copy(
                src_ref=xsend_buf.at[pl.ds(OFFS[k], CHUNKS[k]), :],
                dst_ref=xrecv_buf.at[pl.ds(OFFS[k], CHUNKS[k]), :],
                send_sem=xsend_sems.at[k],
                recv_sem=xrecv_sems.at[k],
                device_id=xp,
                device_id_type=pl.DeviceIdType.MESH,
            )
            r.start()
            xrdma.append(r)

        loc_dma.wait()

        yrdma = []
        ostore = []
        for k in range(NC):
            sl = pl.ds(OFFS[k], CHUNKS[k])
            xrdma[k].wait_recv()
            y = loc_buf[sl, :] + xrecv_buf[sl, :]
            ms = jnp.mean(y * y, axis=-1, keepdims=True)
            ystage[sl, :] = y * lax.rsqrt(ms + 1e-6) * g_ref[0, :][None, :]
            osl = pl.ds(out_off + OFFS[k], CHUNKS[k])
            r = pltpu.make_async_remote_copy(
                src_ref=ystage.at[sl, :],
                dst_ref=o_ref.at[osl, :],
                send_sem=ysend_sems.at[k],
                recv_sem=yrecv_sems.at[k],
                device_id=yp,
                device_id_type=pl.DeviceIdType.MESH,
            )
            r.start()
            yrdma.append(r)
            s = pltpu.make_async_copy(
                ystage.at[sl, :], o_ref.at[osl, :], ostore_sems.at[k]
            )
            s.start()
            ostore.append(s)

        for k in range(NC):
            yrdma[k].wait_recv()
            xrdma[k].wait_send()
            yrdma[k].wait_send()
            ostore[k].wait()

    out = pl.pallas_call(
        body,
        out_shape=jax.ShapeDtypeStruct((m_half, d), jnp.float32),
        in_specs=[
            pl.BlockSpec(memory_space=pl.ANY),
            pl.BlockSpec(memory_space=pltpu.VMEM),
        ],
        out_specs=pl.BlockSpec(memory_space=pl.ANY),
        scratch_shapes=[
            pltpu.VMEM((m_q, d), jnp.float32),
            pltpu.VMEM((m_q, d), jnp.float32),
            pltpu.VMEM((m_q, d), jnp.float32),
            pltpu.VMEM((m_q, d), jnp.float32),
            pltpu.SemaphoreType.DMA,
            pltpu.SemaphoreType.DMA((NC,)),
            pltpu.SemaphoreType.DMA((NC,)),
            pltpu.SemaphoreType.DMA((NC,)),
            pltpu.SemaphoreType.DMA((NC,)),
            pltpu.SemaphoreType.DMA((NC,)),
            pltpu.SemaphoreType.DMA((NC,)),
        ],
        compiler_params=pltpu.CompilerParams(collective_id=0),
    )(p2d, g2d)
    return out
